# Initial kernel scaffold; baseline (speedup 1.0000x reference)
import jax
import jax.numpy as jnp
from jax.experimental import pallas as pl


def kernel(x, edge_index, batch, W_emb, ggc_w, w_ih, w_hh, b_ih, b_hh, W1, b1, W2, b2):
    raise NotImplementedError("write your pallas kernel here")


# speedup vs baseline: 3.8050x; 3.8050x over previous
import functools

import jax
import jax.numpy as jnp
from jax import lax
from jax.experimental import pallas as pl
from jax.experimental.pallas import tpu as pltpu
from jax.experimental.pallas import tpu_sc as plsc

_N = 10000
_E = 320000
_H = 128
_G = 256
_STEPS = 4

_NC = 2
_NS = 16
_NW = _NC * _NS
_CHUNK = 128
_NCH = 79
_EPT = _NCH * _CHUNK
_EPAD = _NW * _EPT
_R = 10240
_RPT = _R // _NS

_BLK = 2000
_NBLK = _N // _BLK


def _sc_scatter_body(m_hbm, zero_hbm, src_hbm, dst_hbm, out_hbm,
                     agg_sh, src_v, dst_v, rows_v, sem):
    c = lax.axis_index("c")
    s = lax.axis_index("s")
    wid = c * _NS + s
    pltpu.sync_copy(zero_hbm.at[pl.ds(s * _RPT, _RPT)],
                    agg_sh.at[pl.ds(s * _RPT, _RPT)])
    pltpu.sync_copy(src_hbm.at[wid], src_v)
    pltpu.sync_copy(dst_hbm.at[wid], dst_v)
    plsc.subcore_barrier()

    @pl.loop(0, _NCH)
    def _chunk(j):
        pltpu.async_copy(m_hbm.at[src_v.at[j]], rows_v, sem).wait()
        pltpu.sync_copy(rows_v, agg_sh.at[dst_v.at[j]], add=True)

    plsc.subcore_barrier()
    pltpu.sync_copy(agg_sh.at[pl.ds(s * _RPT, _RPT)],
                    out_hbm.at[c, pl.ds(s * _RPT, _RPT)])


_sc_scatter = pl.kernel(
    _sc_scatter_body,
    out_type=jax.ShapeDtypeStruct((_NC, _R, _H), jnp.float32),
    mesh=plsc.VectorSubcoreMesh(core_axis_name="c", subcore_axis_name="s"),
    scratch_types=[
        pltpu.VMEM_SHARED((_R, _H), jnp.float32),
        pltpu.VMEM((_NCH, _CHUNK), jnp.int32),
        pltpu.VMEM((_NCH, _CHUNK), jnp.int32),
        pltpu.VMEM((_CHUNK, _H), jnp.float32),
        pltpu.SemaphoreType.DMA,
    ],
)


def _embed_body(x_ref, wemb_ref, w0_ref, h_ref, m_ref):
    h = jnp.maximum(
        jnp.dot(x_ref[...], wemb_ref[...], preferred_element_type=jnp.float32),
        0.0)
    h_ref[...] = h
    m_ref[...] = jnp.dot(h, w0_ref[...], preferred_element_type=jnp.float32)


_embed = pl.pallas_call(
    _embed_body,
    grid=(_NBLK,),
    in_specs=[
        pl.BlockSpec((_BLK, _H), lambda i: (i, 0)),
        pl.BlockSpec((_H, _H), lambda i: (0, 0)),
        pl.BlockSpec((_H, _H), lambda i: (0, 0)),
    ],
    out_specs=[
        pl.BlockSpec((_BLK, _H), lambda i: (i, 0)),
        pl.BlockSpec((_BLK, _H), lambda i: (i, 0)),
    ],
    out_shape=[
        jax.ShapeDtypeStruct((_N, _H), jnp.float32),
        jax.ShapeDtypeStruct((_N, _H), jnp.float32),
    ],
)


def _gru_body(last, p0_ref, p1_ref, h_ref, wih_ref, whh_ref, bih_ref,
              bhh_ref, wn_ref, h_out, m_out=None):
    agg = p0_ref[...] + p1_ref[...]
    gi = jnp.dot(agg, wih_ref[...],
                 preferred_element_type=jnp.float32) + bih_ref[...]
    gh = jnp.dot(h_ref[...], whh_ref[...],
                 preferred_element_type=jnp.float32) + bhh_ref[...]
    r = jax.nn.sigmoid(gi[:, :_H] + gh[:, :_H])
    z = jax.nn.sigmoid(gi[:, _H:2 * _H] + gh[:, _H:2 * _H])
    n = jnp.tanh(gi[:, 2 * _H:] + r * gh[:, 2 * _H:])
    h_new = (1.0 - z) * n + z * h_ref[...]
    if last:
        h_out[...] = jnp.maximum(h_new, 0.0)
    else:
        h_out[...] = h_new
        m_out[...] = jnp.dot(h_new, wn_ref[...],
                             preferred_element_type=jnp.float32)


def _make_gru(last):
    n_out = 1 if last else 2
    return pl.pallas_call(
        functools.partial(_gru_body, last),
        grid=(_NBLK,),
        in_specs=[
            pl.BlockSpec((_BLK, _H), lambda i: (i, 0)),
            pl.BlockSpec((_BLK, _H), lambda i: (i, 0)),
            pl.BlockSpec((_BLK, _H), lambda i: (i, 0)),
            pl.BlockSpec((_H, 3 * _H), lambda i: (0, 0)),
            pl.BlockSpec((_H, 3 * _H), lambda i: (0, 0)),
            pl.BlockSpec((1, 3 * _H), lambda i: (0, 0)),
            pl.BlockSpec((1, 3 * _H), lambda i: (0, 0)),
            pl.BlockSpec((_H, _H), lambda i: (0, 0)),
        ],
        out_specs=[pl.BlockSpec((_BLK, _H), lambda i: (i, 0))] * n_out,
        out_shape=[jax.ShapeDtypeStruct((_N, _H), jnp.float32)] * n_out,
    )


_gru_mid = _make_gru(False)
_gru_last = _make_gru(True)


def _pool_body(h_ref, batch_ref, w1_ref, b1_ref, w2_ref, b2_ref,
               out_ref, sums_sc, counts_sc):
    i = pl.program_id(0)

    @pl.when(i == 0)
    def _():
        sums_sc[...] = jnp.zeros_like(sums_sc)
        counts_sc[...] = jnp.zeros_like(counts_sc)

    seg = lax.broadcasted_iota(jnp.int32, (_G, _BLK), 0)
    onehot = (seg == batch_ref[0]).astype(jnp.float32)
    sums_sc[...] += jnp.dot(onehot, h_ref[...],
                            preferred_element_type=jnp.float32)
    counts_sc[...] += jnp.sum(onehot, axis=1, keepdims=True)

    @pl.when(i == _NBLK - 1)
    def _():
        pooled = sums_sc[...] / jnp.maximum(counts_sc[...], 1.0)
        y = jnp.maximum(
            jnp.dot(pooled, w1_ref[...],
                    preferred_element_type=jnp.float32) + b1_ref[...], 0.0)
        o = jnp.dot(y, w2_ref[...],
                    preferred_element_type=jnp.float32) + b2_ref[...]
        out_ref[...] = jax.nn.softplus(o)


_pool = pl.pallas_call(
    _pool_body,
    grid=(_NBLK,),
    in_specs=[
        pl.BlockSpec((_BLK, _H), lambda i: (i, 0)),
        pl.BlockSpec((1, 1, _BLK), lambda i: (i, 0, 0)),
        pl.BlockSpec((_H, _H), lambda i: (0, 0)),
        pl.BlockSpec((1, _H), lambda i: (0, 0)),
        pl.BlockSpec((_H, _H), lambda i: (0, 0)),
        pl.BlockSpec((1, _H), lambda i: (0, 0)),
    ],
    out_specs=pl.BlockSpec((_G, _H), lambda i: (0, 0)),
    out_shape=jax.ShapeDtypeStruct((_G, _H), jnp.float32),
    scratch_shapes=[
        pltpu.VMEM((_G, _H), jnp.float32),
        pltpu.VMEM((_G, 1), jnp.float32),
    ],
)


def kernel(x, edge_index, batch, W_emb, ggc_w, w_ih, w_hh, b_ih, b_hh,
           W1, b1, W2, b2):
    src = edge_index[0].astype(jnp.int32)
    dst = edge_index[1].astype(jnp.int32)
    pad = _EPAD - _E
    src_p = jnp.concatenate([src, jnp.zeros((pad,), jnp.int32)])
    dst_p = jnp.concatenate([dst, jnp.full((pad,), _N, jnp.int32)])
    src3 = src_p.reshape(_NW, _NCH, _CHUNK)
    dst3 = dst_p.reshape(_NW, _NCH, _CHUNK)
    zeros = jnp.zeros((_R, _H), jnp.float32)

    wih_t = w_ih.T
    whh_t = w_hh.T
    bih2 = b_ih.reshape(1, 3 * _H)
    bhh2 = b_hh.reshape(1, 3 * _H)
    w2p = jnp.pad(W2, ((0, 0), (0, _H - W2.shape[1])))
    b2b = jnp.broadcast_to(b2, (1, _H))
    b12 = b1.reshape(1, _H)
    batch3 = batch.astype(jnp.int32).reshape(_NBLK, 1, _BLK)

    h, m = _embed(x, W_emb, ggc_w[0])
    for i in range(_STEPS):
        partial = _sc_scatter(m, zeros, src3, dst3)
        last = i == _STEPS - 1
        wn = ggc_w[0] if last else ggc_w[i + 1]
        if last:
            (h,) = _gru_last(partial[0], partial[1], h, wih_t, whh_t,
                             bih2, bhh2, wn)
        else:
            h, m = _gru_mid(partial[0], partial[1], h, wih_t, whh_t,
                            bih2, bhh2, wn)
    out = _pool(h, batch3, W1, b12, w2p, b2b)
    return out[:, 0]
